# Initial kernel scaffold; baseline (speedup 1.0000x reference)
import jax
import jax.numpy as jnp
from jax.experimental import pallas as pl


def kernel(x, w1, b1, w2, b2, wf1, bf1, wf2, bf2):
    raise NotImplementedError("write your pallas kernel here")


# speedup vs baseline: 148.1930x; 148.1930x over previous
import jax
import jax.numpy as jnp
from jax.experimental import pallas as pl
from jax.experimental.pallas import tpu as pltpu


def _params_for_tc():
    cls = getattr(pltpu, "CompilerParams", None)
    if cls is None:
        cls = getattr(pltpu, "TPUCompilerParams", None)
    if cls is None:
        return None
    return cls(dimension_semantics=("parallel",))


def _build_w1row(w1):
    iw = jnp.arange(28)[:, None]
    ow = jnp.arange(24)[None, :]
    d = iw - ow
    mask = (d >= 0) & (d < 5)
    g = jnp.take(w1, jnp.clip(d, 0, 4), axis=1)
    g = g * mask[None, :, :, None]
    return g.reshape(140, 240)


def _build_w2band(w2):
    w2r = w2.reshape(5, 5, 10, 20)
    iw = jnp.arange(12)[:, None]
    ow = jnp.arange(8)[None, :]
    d = iw - ow
    mask = (d >= 0) & (d < 5)
    g = jnp.take(w2r, jnp.clip(d, 0, 4).reshape(-1), axis=1)
    g = g.reshape(5, 12, 8, 10, 20) * mask[None, :, :, None, None]
    g = jnp.transpose(g, (0, 1, 3, 2, 4))
    g = g.reshape(5, 12, 10, 160)
    g = jnp.pad(g, ((0, 0), (0, 0), (0, 10), (0, 0)))
    g = g.reshape(5, 240, 160)
    g = jnp.pad(g, ((0, 0), (0, 16), (0, 0)))
    return g.reshape(1280, 160)


def _build_wf1(wf1):
    g = wf1.reshape(4, 4, 20, 50)
    g = jnp.pad(g, ((0, 0), (0, 0), (0, 20), (0, 0)))
    g = g.reshape(4, 160, 50)
    g = jnp.pad(g, ((0, 0), (0, 96), (0, 0)))
    return g.reshape(1024, 50)


def _build_b1row(b1):
    return jnp.tile(jnp.pad(b1, ((0, 0), (0, 10))), (1, 12))


def _build_b2row(b2):
    return jnp.tile(jnp.pad(b2, ((0, 0), (0, 20))), (1, 4))


def _fused_body(x_ref, w1_ref, b1_ref, w2_ref, b2_ref,
                wf1_ref, bf1_ref, wf2_ref, bf2_ref,
                out_ref, h1_ref, pool_ref):
    f32 = jnp.float32
    X = x_ref[...]
    w1 = w1_ref[...]
    b1 = b1_ref[...]
    h1_ref[...] = jnp.zeros_like(h1_ref)
    pool_ref[...] = jnp.zeros_like(pool_ref)

    for hh in range(12):
        base = 28 * (2 * hh)
        y0 = jnp.dot(X[:, base:base + 140], w1, preferred_element_type=f32)
        y1 = jnp.dot(X[:, base + 28:base + 168], w1, preferred_element_type=f32)
        m = jnp.maximum(y0, y1)
        z = jnp.maximum(m[:, :230], m[:, 10:240])
        h1_ref[:, 256 * hh:256 * hh + 230] = jnp.maximum(z + b1[:, :230], 0.0)

    w2 = w2_ref[...]
    b2 = b2_ref[...]
    for ii in range(4):
        base = 256 * (2 * ii)
        a0 = jnp.dot(h1_ref[:, base:base + 1280], w2,
                     preferred_element_type=f32)
        a1 = jnp.dot(h1_ref[:, base + 256:base + 1536], w2,
                     preferred_element_type=f32)
        m = jnp.maximum(a0, a1)
        z = jnp.maximum(m[:, :140], m[:, 20:160])
        pool_ref[:, 256 * ii:256 * ii + 140] = jnp.maximum(z + b2[:, :140], 0.0)

    h3 = jnp.dot(pool_ref[...], wf1_ref[...], preferred_element_type=f32)
    h3 = jnp.maximum(h3 + bf1_ref[...], 0.0)
    logits = jnp.dot(h3, wf2_ref[...], preferred_element_type=f32) + bf2_ref[...]
    mx = jnp.max(logits, axis=-1, keepdims=True)
    e = jnp.exp(logits - mx)
    out_ref[...] = logits - mx - jnp.log(jnp.sum(e, axis=-1, keepdims=True))


def kernel(x, w1, b1, w2, b2, wf1, bf1, wf2, bf2):
    N = x.shape[0]
    xf = x.reshape(N, 784).astype(jnp.float32)
    B = 1
    for cand in (256, 128, 64, 32, 16, 8, 4, 2):
        if N % cand == 0:
            B = cand
            break

    w1row = _build_w1row(w1)
    w2band = _build_w2band(w2)
    wf1b = _build_wf1(wf1)
    b1row = _build_b1row(b1)
    b2row = _build_b2row(b2)

    grid_spec = pltpu.PrefetchScalarGridSpec(
        num_scalar_prefetch=0,
        grid=(N // B,),
        in_specs=[
            pl.BlockSpec((B, 784), lambda n: (n, 0)),
            pl.BlockSpec((140, 240), lambda n: (0, 0)),
            pl.BlockSpec((1, 240), lambda n: (0, 0)),
            pl.BlockSpec((1280, 160), lambda n: (0, 0)),
            pl.BlockSpec((1, 160), lambda n: (0, 0)),
            pl.BlockSpec((1024, 50), lambda n: (0, 0)),
            pl.BlockSpec((1, 50), lambda n: (0, 0)),
            pl.BlockSpec((50, 10), lambda n: (0, 0)),
            pl.BlockSpec((1, 10), lambda n: (0, 0)),
        ],
        out_specs=pl.BlockSpec((B, 10), lambda n: (n, 0)),
        scratch_shapes=[
            pltpu.VMEM((B, 12 * 256), jnp.float32),
            pltpu.VMEM((B, 4 * 256), jnp.float32),
        ],
    )
    return pl.pallas_call(
        _fused_body,
        out_shape=jax.ShapeDtypeStruct((N, 10), jnp.float32),
        grid_spec=grid_spec,
        compiler_params=_params_for_tc(),
    )(xf, w1row, b1row, w2band, b2row, wf1b, bf1, wf2, bf2)


# speedup vs baseline: 217.1811x; 1.4655x over previous
import jax
import jax.numpy as jnp
from jax.experimental import pallas as pl
from jax.experimental.pallas import tpu as pltpu


def _params_for_tc():
    cls = getattr(pltpu, "CompilerParams", None)
    if cls is None:
        cls = getattr(pltpu, "TPUCompilerParams", None)
    if cls is None:
        return None
    return cls(dimension_semantics=("parallel",))


def _build_w1(w1):
    c = jnp.arange(28)[:, None]
    ow = jnp.arange(24)[None, :]
    d = c - ow
    mask = (d >= 0) & (d < 5)
    g = jnp.take(w1, jnp.clip(d, 0, 4), axis=1)
    g = g * mask[None, :, :, None]
    g = g.reshape(5, 28, 12, 2, 10)
    g = jnp.transpose(g, (3, 4, 2, 0, 1))
    w1a = jnp.pad(g.reshape(240, 140), ((0, 0), (0, 28)))
    w1b = jnp.concatenate([jnp.zeros((240, 28), w1a.dtype),
                           w1a[:, :140]], axis=1)
    return w1a, w1b


def _build_w2(w2):
    w2r = w2.reshape(5, 5, 10, 20)
    iw = jnp.arange(12)[:, None]
    ow = jnp.arange(8)[None, :]
    d = iw - ow
    mask = (d >= 0) & (d < 5)
    g = jnp.take(w2r, jnp.clip(d, 0, 4).reshape(-1), axis=1)
    g = g.reshape(5, 12, 8, 10, 20) * mask[None, :, :, None, None]
    g = g.reshape(5, 12, 4, 2, 10, 20)
    g = jnp.transpose(g, (3, 5, 2, 0, 4, 1))
    w2a = jnp.pad(g.reshape(160, 600), ((0, 0), (0, 120)))
    w2b = jnp.concatenate([jnp.zeros((160, 120), w2a.dtype),
                           w2a[:, :600]], axis=1)
    return w2a, w2b


def _build_wf1(wf1):
    g = wf1.reshape(4, 4, 20, 50)
    g = jnp.transpose(g, (3, 0, 2, 1))
    return g.reshape(50, 320)


def _fused_body(x_ref, w1a_ref, w1b_ref, b1_ref, w2a_ref, w2b_ref, b2_ref,
                wf1_ref, bf1_ref, wf2_ref, bf2_ref,
                out_ref, h1_ref, p_ref):
    f32 = jnp.float32
    X = x_ref[...]
    w1a = w1a_ref[...]
    w1b = w1b_ref[...]
    b1t = b1_ref[...]

    for h in range(12):
        X6 = X[56 * h:56 * h + 168, :]
        ya = jnp.dot(w1a, X6, preferred_element_type=f32)
        yb = jnp.dot(w1b, X6, preferred_element_type=f32)
        m = jnp.maximum(ya, yb)
        z = jnp.maximum(m[:120, :], m[120:240, :])
        h1_ref[120 * h:120 * h + 120, :] = jnp.maximum(z + b1t, 0.0)

    w2a = w2a_ref[...]
    w2b = w2b_ref[...]
    b2t = b2_ref[...]
    for i in range(4):
        H6 = h1_ref[240 * i:240 * i + 720, :]
        ya = jnp.dot(w2a, H6, preferred_element_type=f32)
        yb = jnp.dot(w2b, H6, preferred_element_type=f32)
        m = jnp.maximum(ya, yb)
        z = jnp.maximum(m[:80, :], m[80:160, :])
        p_ref[80 * i:80 * i + 80, :] = jnp.maximum(z + b2t, 0.0)

    h3 = jnp.dot(wf1_ref[...], p_ref[...], preferred_element_type=f32)
    h3 = jnp.maximum(h3 + bf1_ref[...], 0.0)
    logits = jnp.dot(wf2_ref[...], h3, preferred_element_type=f32) + bf2_ref[...]
    mx = jnp.max(logits, axis=0, keepdims=True)
    e = jnp.exp(logits - mx)
    out_ref[...] = logits - mx - jnp.log(jnp.sum(e, axis=0, keepdims=True))


def kernel(x, w1, b1, w2, b2, wf1, bf1, wf2, bf2):
    N = x.shape[0]
    xt = jnp.transpose(x.reshape(N, 784), (1, 0))
    BL = 8
    for cand in (512, 256, 128, 64, 32, 16):
        if N % cand == 0:
            BL = cand
            break

    w1a, w1b = _build_w1(w1)
    w2a, w2b = _build_w2(w2)
    wf1b = _build_wf1(wf1)
    b1t = jnp.broadcast_to(b1.reshape(10, 1, 1), (10, 12, BL)).reshape(120, BL)
    b2t = jnp.broadcast_to(b2.reshape(20, 1, 1), (20, 4, BL)).reshape(80, BL)
    bf1t = jnp.broadcast_to(bf1.reshape(50, 1), (50, BL))
    bf2t = jnp.broadcast_to(bf2.reshape(10, 1), (10, BL))
    wf2t = wf2.T

    grid_spec = pltpu.PrefetchScalarGridSpec(
        num_scalar_prefetch=0,
        grid=(N // BL,),
        in_specs=[
            pl.BlockSpec((784, BL), lambda n: (0, n)),
            pl.BlockSpec((240, 168), lambda n: (0, 0)),
            pl.BlockSpec((240, 168), lambda n: (0, 0)),
            pl.BlockSpec((120, BL), lambda n: (0, 0)),
            pl.BlockSpec((160, 720), lambda n: (0, 0)),
            pl.BlockSpec((160, 720), lambda n: (0, 0)),
            pl.BlockSpec((80, BL), lambda n: (0, 0)),
            pl.BlockSpec((50, 320), lambda n: (0, 0)),
            pl.BlockSpec((50, BL), lambda n: (0, 0)),
            pl.BlockSpec((10, 50), lambda n: (0, 0)),
            pl.BlockSpec((10, BL), lambda n: (0, 0)),
        ],
        out_specs=pl.BlockSpec((10, BL), lambda n: (0, n)),
        scratch_shapes=[
            pltpu.VMEM((1440, BL), jnp.float32),
            pltpu.VMEM((320, BL), jnp.float32),
        ],
    )
    out = pl.pallas_call(
        _fused_body,
        out_shape=jax.ShapeDtypeStruct((10, N), jnp.float32),
        grid_spec=grid_spec,
        compiler_params=_params_for_tc(),
    )(xt, w1a, w1b, b1t, w2a, w2b, b2t, wf1b, bf1t, wf2t, bf2t)
    return out.T


# speedup vs baseline: 471.9386x; 2.1730x over previous
import jax
import jax.numpy as jnp
from jax.experimental import pallas as pl
from jax.experimental.pallas import tpu as pltpu


def _params_for_tc():
    cls = getattr(pltpu, "CompilerParams", None)
    if cls is None:
        cls = getattr(pltpu, "TPUCompilerParams", None)
    if cls is None:
        return None
    return cls(dimension_semantics=("parallel",))


def _build_w1(w1):
    c = jnp.arange(28)[:, None]
    ow = jnp.arange(24)[None, :]
    d = c - ow
    mask = (d >= 0) & (d < 5)
    g = jnp.take(w1, jnp.clip(d, 0, 4), axis=1)
    g = g * mask[None, :, :, None]
    g = g.reshape(5, 28, 12, 2, 10)
    g = jnp.transpose(g, (3, 4, 2, 0, 1))
    w1a = jnp.pad(g.reshape(240, 140), ((0, 0), (0, 28)))
    w1b = jnp.concatenate([jnp.zeros((240, 28), w1a.dtype),
                           w1a[:, :140]], axis=1)
    return w1a, w1b


def _build_w2(w2):
    w2r = w2.reshape(5, 5, 10, 20)
    iw = jnp.arange(12)[:, None]
    ow = jnp.arange(8)[None, :]
    d = iw - ow
    mask = (d >= 0) & (d < 5)
    g = jnp.take(w2r, jnp.clip(d, 0, 4).reshape(-1), axis=1)
    g = g.reshape(5, 12, 8, 10, 20) * mask[None, :, :, None, None]
    g = g.reshape(5, 12, 4, 2, 10, 20)
    g = jnp.transpose(g, (3, 5, 2, 0, 4, 1))
    w2a = jnp.pad(g.reshape(160, 600), ((0, 0), (0, 120)))
    w2b = jnp.concatenate([jnp.zeros((160, 120), w2a.dtype),
                           w2a[:, :600]], axis=1)
    return w2a, w2b


def _build_wf1(wf1):
    g = wf1.reshape(4, 4, 20, 50)
    g = jnp.transpose(g, (3, 0, 2, 1))
    return g.reshape(50, 320)


def _make_body(BL):
    CB = BL // 128

    def _fused_body(x_ref, w1a_ref, w1b_ref, b1_ref, w2a_ref, w2b_ref,
                    b2_ref, wf1_ref, bf1_ref, wf2_ref, bf2_ref,
                    out_ref, xs_ref, h1_ref, p_ref):
        f32 = jnp.float32
        X3 = x_ref[...]
        xs_ref[...] = jnp.swapaxes(
            X3.reshape(98, 8, CB, 128), 1, 2).reshape(784, BL)
        X = xs_ref[...]
        w1a = w1a_ref[...]
        w1b = w1b_ref[...]
        b1t = b1_ref[...]

        for h in range(12):
            X6 = X[56 * h:56 * h + 168, :]
            ya = jnp.dot(w1a, X6, preferred_element_type=f32)
            yb = jnp.dot(w1b, X6, preferred_element_type=f32)
            m = jnp.maximum(ya, yb)
            z = jnp.maximum(m[:120, :], m[120:240, :])
            h1_ref[120 * h:120 * h + 120, :] = jnp.maximum(z + b1t, 0.0)

        w2a = w2a_ref[...]
        w2b = w2b_ref[...]
        b2t = b2_ref[...]
        for i in range(4):
            H6 = h1_ref[240 * i:240 * i + 720, :]
            ya = jnp.dot(w2a, H6, preferred_element_type=f32)
            yb = jnp.dot(w2b, H6, preferred_element_type=f32)
            m = jnp.maximum(ya, yb)
            z = jnp.maximum(m[:80, :], m[80:160, :])
            p_ref[80 * i:80 * i + 80, :] = jnp.maximum(z + b2t, 0.0)

        h3 = jnp.dot(wf1_ref[...], p_ref[...], preferred_element_type=f32)
        h3 = jnp.maximum(h3 + bf1_ref[...], 0.0)
        logits = (jnp.dot(wf2_ref[...], h3, preferred_element_type=f32)
                  + bf2_ref[...])
        mx = jnp.max(logits, axis=0, keepdims=True)
        e = jnp.exp(logits - mx)
        out_ref[...] = logits - mx - jnp.log(jnp.sum(e, axis=0, keepdims=True))

    return _fused_body


def kernel(x, w1, b1, w2, b2, wf1, bf1, wf2, bf2):
    N = x.shape[0]
    Np = (N + 127) // 128 * 128
    xp = jnp.pad(x, ((0, Np - N), (0, 0), (0, 0), (0, 0))) if Np != N else x
    xv = jnp.transpose(xp, (1, 2, 3, 0)).reshape(784, Np // 128, 128)

    BL = 128
    for cand in (1024, 512, 256):
        if Np % cand == 0:
            BL = cand
            break
    CB = BL // 128

    w1a, w1b = _build_w1(w1)
    w2a, w2b = _build_w2(w2)
    wf1b = _build_wf1(wf1)
    b1t = jnp.broadcast_to(b1.reshape(10, 1, 1), (10, 12, BL)).reshape(120, BL)
    b2t = jnp.broadcast_to(b2.reshape(20, 1, 1), (20, 4, BL)).reshape(80, BL)
    bf1t = jnp.broadcast_to(bf1.reshape(50, 1), (50, BL))
    bf2t = jnp.broadcast_to(bf2.reshape(10, 1), (10, BL))
    wf2t = wf2.T

    grid_spec = pltpu.PrefetchScalarGridSpec(
        num_scalar_prefetch=0,
        grid=(Np // BL,),
        in_specs=[
            pl.BlockSpec((784, CB, 128), lambda n: (0, n, 0)),
            pl.BlockSpec((240, 168), lambda n: (0, 0)),
            pl.BlockSpec((240, 168), lambda n: (0, 0)),
            pl.BlockSpec((120, BL), lambda n: (0, 0)),
            pl.BlockSpec((160, 720), lambda n: (0, 0)),
            pl.BlockSpec((160, 720), lambda n: (0, 0)),
            pl.BlockSpec((80, BL), lambda n: (0, 0)),
            pl.BlockSpec((50, 320), lambda n: (0, 0)),
            pl.BlockSpec((50, BL), lambda n: (0, 0)),
            pl.BlockSpec((10, 50), lambda n: (0, 0)),
            pl.BlockSpec((10, BL), lambda n: (0, 0)),
        ],
        out_specs=pl.BlockSpec((10, BL), lambda n: (0, n)),
        scratch_shapes=[
            pltpu.VMEM((784, BL), jnp.float32),
            pltpu.VMEM((1440, BL), jnp.float32),
            pltpu.VMEM((320, BL), jnp.float32),
        ],
    )
    out = pl.pallas_call(
        _make_body(BL),
        out_shape=jax.ShapeDtypeStruct((10, Np), jnp.float32),
        grid_spec=grid_spec,
        compiler_params=_params_for_tc(),
    )(xv, w1a, w1b, b1t, w2a, w2b, b2t, wf1b, bf1t, wf2t, bf2t)
    return out[:, :N].T if Np != N else out.T


# speedup vs baseline: 474.2514x; 1.0049x over previous
import jax
import jax.numpy as jnp
from jax.experimental import pallas as pl
from jax.experimental.pallas import tpu as pltpu


def _params_for_tc():
    cls = getattr(pltpu, "CompilerParams", None)
    if cls is None:
        cls = getattr(pltpu, "TPUCompilerParams", None)
    if cls is None:
        return None
    return cls(dimension_semantics=("parallel",))


def _build_w1(w1):
    c = jnp.arange(28)[:, None]
    ow = jnp.arange(24)[None, :]
    d = c - ow
    mask = (d >= 0) & (d < 5)
    g = jnp.take(w1, jnp.clip(d, 0, 4), axis=1)
    g = g * mask[None, :, :, None]
    g = g.reshape(5, 28, 12, 2, 10)
    g = jnp.transpose(g, (3, 4, 2, 0, 1))
    w1a = jnp.pad(g.reshape(240, 140), ((0, 0), (0, 28)))
    w1b = jnp.concatenate([jnp.zeros((240, 28), w1a.dtype),
                           w1a[:, :140]], axis=1)
    return w1a, w1b


def _build_w2(w2):
    w2r = w2.reshape(5, 5, 10, 20)
    iw = jnp.arange(12)[:, None]
    ow = jnp.arange(8)[None, :]
    d = iw - ow
    mask = (d >= 0) & (d < 5)
    g = jnp.take(w2r, jnp.clip(d, 0, 4).reshape(-1), axis=1)
    g = g.reshape(5, 12, 8, 10, 20) * mask[None, :, :, None, None]
    g = g.reshape(5, 12, 4, 2, 10, 20)
    g = jnp.transpose(g, (3, 5, 2, 0, 4, 1))
    w2a = jnp.pad(g.reshape(160, 600), ((0, 0), (0, 120)))
    w2b = jnp.concatenate([jnp.zeros((160, 120), w2a.dtype),
                           w2a[:, :600]], axis=1)
    return w2a, w2b


def _build_wf1(wf1):
    g = wf1.reshape(4, 4, 20, 50)
    g = jnp.transpose(g, (3, 0, 2, 1))
    return g.reshape(50, 320)


def _make_body(BL):
    CB = BL // 128

    def _fused_body(x_ref, w1a_ref, w1b_ref, b1_ref, w2a_ref, w2b_ref,
                    b2_ref, wf1_ref, bf1_ref, wf2_ref, bf2_ref,
                    out_ref, xs_ref, h1_ref, p_ref):
        f32 = jnp.float32
        X3 = x_ref[...]
        xs_ref[...] = X3.reshape(784, BL)
        X = xs_ref[...]
        w1a = w1a_ref[...]
        w1b = w1b_ref[...]
        b1t = b1_ref[...]

        for h in range(12):
            X6 = X[56 * h:56 * h + 168, :]
            ya = jnp.dot(w1a, X6, preferred_element_type=f32)
            yb = jnp.dot(w1b, X6, preferred_element_type=f32)
            m = jnp.maximum(ya, yb)
            z = jnp.maximum(m[:120, :], m[120:240, :])
            h1_ref[120 * h:120 * h + 120, :] = jnp.maximum(z + b1t, 0.0)

        w2a = w2a_ref[...]
        w2b = w2b_ref[...]
        b2t = b2_ref[...]
        for i in range(4):
            H6 = h1_ref[240 * i:240 * i + 720, :]
            ya = jnp.dot(w2a, H6, preferred_element_type=f32)
            yb = jnp.dot(w2b, H6, preferred_element_type=f32)
            m = jnp.maximum(ya, yb)
            z = jnp.maximum(m[:80, :], m[80:160, :])
            p_ref[80 * i:80 * i + 80, :] = jnp.maximum(z + b2t, 0.0)

        h3 = jnp.dot(wf1_ref[...], p_ref[...], preferred_element_type=f32)
        h3 = jnp.maximum(h3 + bf1_ref[...], 0.0)
        logits = (jnp.dot(wf2_ref[...], h3, preferred_element_type=f32)
                  + bf2_ref[...])
        mx = jnp.max(logits, axis=0, keepdims=True)
        e = jnp.exp(logits - mx)
        out_ref[...] = logits - mx - jnp.log(jnp.sum(e, axis=0, keepdims=True))

    return _fused_body


def kernel(x, w1, b1, w2, b2, wf1, bf1, wf2, bf2):
    N = x.shape[0]
    Np = (N + 127) // 128 * 128
    xp = jnp.pad(x, ((0, Np - N), (0, 0), (0, 0), (0, 0))) if Np != N else x
    xv = jnp.transpose(xp, (1, 2, 3, 0)).reshape(784, Np // 128, 128)

    BL = 128
    for cand in (1024, 512, 256):
        if Np % cand == 0:
            BL = cand
            break
    CB = BL // 128

    w1a, w1b = _build_w1(w1)
    w2a, w2b = _build_w2(w2)
    wf1b = _build_wf1(wf1)
    b1t = jnp.broadcast_to(b1.reshape(10, 1, 1), (10, 12, BL)).reshape(120, BL)
    b2t = jnp.broadcast_to(b2.reshape(20, 1, 1), (20, 4, BL)).reshape(80, BL)
    bf1t = jnp.broadcast_to(bf1.reshape(50, 1), (50, BL))
    bf2t = jnp.broadcast_to(bf2.reshape(10, 1), (10, BL))
    wf2t = wf2.T

    grid_spec = pltpu.PrefetchScalarGridSpec(
        num_scalar_prefetch=0,
        grid=(Np // BL,),
        in_specs=[
            pl.BlockSpec((784, CB, 128), lambda n: (0, n, 0)),
            pl.BlockSpec((240, 168), lambda n: (0, 0)),
            pl.BlockSpec((240, 168), lambda n: (0, 0)),
            pl.BlockSpec((120, BL), lambda n: (0, 0)),
            pl.BlockSpec((160, 720), lambda n: (0, 0)),
            pl.BlockSpec((160, 720), lambda n: (0, 0)),
            pl.BlockSpec((80, BL), lambda n: (0, 0)),
            pl.BlockSpec((50, 320), lambda n: (0, 0)),
            pl.BlockSpec((50, BL), lambda n: (0, 0)),
            pl.BlockSpec((10, 50), lambda n: (0, 0)),
            pl.BlockSpec((10, BL), lambda n: (0, 0)),
        ],
        out_specs=pl.BlockSpec((10, BL), lambda n: (0, n)),
        scratch_shapes=[
            pltpu.VMEM((784, BL), jnp.float32),
            pltpu.VMEM((1440, BL), jnp.float32),
            pltpu.VMEM((320, BL), jnp.float32),
        ],
    )
    out = pl.pallas_call(
        _make_body(BL),
        out_shape=jax.ShapeDtypeStruct((10, Np), jnp.float32),
        grid_spec=grid_spec,
        compiler_params=_params_for_tc(),
    )(xv, w1a, w1b, b1t, w2a, w2b, b2t, wf1b, bf1t, wf2t, bf2t)
    return out[:, :N].T if Np != N else out.T


# speedup vs baseline: 497.5765x; 1.0492x over previous
import jax
import jax.numpy as jnp
from jax.experimental import pallas as pl
from jax.experimental.pallas import tpu as pltpu


def _params_for_tc():
    cls = getattr(pltpu, "CompilerParams", None)
    if cls is None:
        cls = getattr(pltpu, "TPUCompilerParams", None)
    if cls is None:
        return None
    return cls(dimension_semantics=("parallel",))


def _build_w1(w1):
    c = jnp.arange(28)[:, None]
    ow = jnp.arange(24)[None, :]
    d = c - ow
    mask = (d >= 0) & (d < 5)
    g = jnp.take(w1, jnp.clip(d, 0, 4), axis=1)
    g = g * mask[None, :, :, None]
    g = g.reshape(5, 28, 12, 2, 10)
    g = jnp.transpose(g, (3, 4, 2, 0, 1))
    w1a = jnp.pad(g.reshape(240, 140), ((0, 0), (0, 28)))
    w1b = jnp.concatenate([jnp.zeros((240, 28), w1a.dtype),
                           w1a[:, :140]], axis=1)
    return jnp.concatenate([w1a, w1b], axis=0)


def _build_w2(w2):
    w2r = w2.reshape(5, 5, 10, 20)
    iw = jnp.arange(12)[:, None]
    ow = jnp.arange(8)[None, :]
    d = iw - ow
    mask = (d >= 0) & (d < 5)
    g = jnp.take(w2r, jnp.clip(d, 0, 4).reshape(-1), axis=1)
    g = g.reshape(5, 12, 8, 10, 20) * mask[None, :, :, None, None]
    g = g.reshape(5, 12, 4, 2, 10, 20)
    g = jnp.transpose(g, (3, 5, 2, 0, 4, 1))
    w2a = jnp.pad(g.reshape(160, 600), ((0, 0), (0, 120)))
    w2b = jnp.concatenate([jnp.zeros((160, 120), w2a.dtype),
                           w2a[:, :600]], axis=1)
    return jnp.concatenate([w2a, w2b], axis=0)


def _build_wf1(wf1):
    g = wf1.reshape(4, 4, 20, 50)
    g = jnp.transpose(g, (3, 0, 2, 1))
    return g.reshape(50, 320)


def _make_body(BL):
    CB = BL // 128

    def _fused_body(x_ref, w1_ref, b1_ref, w2_ref, b2_ref,
                    wf1_ref, bf1_ref, wf2_ref, bf2_ref,
                    out_ref, xs_ref, h1_ref, p_ref):
        f32 = jnp.float32
        bf16 = jnp.bfloat16
        X3 = x_ref[...]
        xs_ref[...] = X3.reshape(784, BL).astype(bf16)
        X = xs_ref[...]
        w1ab = w1_ref[...]
        b1t = b1_ref[...]

        for h in range(12):
            X6 = X[56 * h:56 * h + 168, :]
            y = jnp.dot(w1ab, X6, preferred_element_type=f32)
            m = jnp.maximum(y[:240, :], y[240:480, :])
            z = jnp.maximum(m[:120, :], m[120:240, :])
            h1_ref[120 * h:120 * h + 120, :] = (
                jnp.maximum(z + b1t, 0.0).astype(bf16))

        w2ab = w2_ref[...]
        b2t = b2_ref[...]
        for i in range(4):
            H6 = h1_ref[240 * i:240 * i + 720, :]
            y = jnp.dot(w2ab, H6, preferred_element_type=f32)
            m = jnp.maximum(y[:160, :], y[160:320, :])
            z = jnp.maximum(m[:80, :], m[80:160, :])
            p_ref[80 * i:80 * i + 80, :] = (
                jnp.maximum(z + b2t, 0.0).astype(bf16))

        h3 = jnp.dot(wf1_ref[...], p_ref[...], preferred_element_type=f32)
        h3 = jnp.maximum(h3 + bf1_ref[...], 0.0).astype(bf16)
        logits = (jnp.dot(wf2_ref[...], h3, preferred_element_type=f32)
                  + bf2_ref[...])
        mx = jnp.max(logits, axis=0, keepdims=True)
        e = jnp.exp(logits - mx)
        out_ref[...] = logits - mx - jnp.log(jnp.sum(e, axis=0, keepdims=True))

    return _fused_body


def kernel(x, w1, b1, w2, b2, wf1, bf1, wf2, bf2):
    N = x.shape[0]
    Np = (N + 127) // 128 * 128
    xp = jnp.pad(x, ((0, Np - N), (0, 0), (0, 0), (0, 0))) if Np != N else x
    xv = jnp.transpose(xp, (1, 2, 3, 0)).reshape(784, Np // 128, 128)

    BL = 128
    for cand in (1024, 512, 256):
        if Np % cand == 0:
            BL = cand
            break
    CB = BL // 128

    bf16 = jnp.bfloat16
    w1ab = _build_w1(w1).astype(bf16)
    w2ab = _build_w2(w2).astype(bf16)
    wf1b = _build_wf1(wf1).astype(bf16)
    b1t = jnp.broadcast_to(b1.reshape(10, 1, 1), (10, 12, BL)).reshape(120, BL)
    b2t = jnp.broadcast_to(b2.reshape(20, 1, 1), (20, 4, BL)).reshape(80, BL)
    bf1t = jnp.broadcast_to(bf1.reshape(50, 1), (50, BL))
    bf2t = jnp.broadcast_to(bf2.reshape(10, 1), (10, BL))
    wf2t = wf2.T.astype(bf16)

    grid_spec = pltpu.PrefetchScalarGridSpec(
        num_scalar_prefetch=0,
        grid=(Np // BL,),
        in_specs=[
            pl.BlockSpec((784, CB, 128), lambda n: (0, n, 0)),
            pl.BlockSpec((480, 168), lambda n: (0, 0)),
            pl.BlockSpec((120, BL), lambda n: (0, 0)),
            pl.BlockSpec((320, 720), lambda n: (0, 0)),
            pl.BlockSpec((80, BL), lambda n: (0, 0)),
            pl.BlockSpec((50, 320), lambda n: (0, 0)),
            pl.BlockSpec((50, BL), lambda n: (0, 0)),
            pl.BlockSpec((10, 50), lambda n: (0, 0)),
            pl.BlockSpec((10, BL), lambda n: (0, 0)),
        ],
        out_specs=pl.BlockSpec((10, BL), lambda n: (0, n)),
        scratch_shapes=[
            pltpu.VMEM((784, BL), jnp.bfloat16),
            pltpu.VMEM((1440, BL), jnp.bfloat16),
            pltpu.VMEM((320, BL), jnp.bfloat16),
        ],
    )
    out = pl.pallas_call(
        _make_body(BL),
        out_shape=jax.ShapeDtypeStruct((10, Np), jnp.float32),
        grid_spec=grid_spec,
        compiler_params=_params_for_tc(),
    )(xv, w1ab, b1t, w2ab, b2t, wf1b, bf1t, wf2t, bf2t)
    return out[:, :N].T if Np != N else out.T


# speedup vs baseline: 503.2267x; 1.0114x over previous
import jax
import jax.numpy as jnp
from jax.experimental import pallas as pl
from jax.experimental.pallas import tpu as pltpu


def _params_for_tc():
    cls = getattr(pltpu, "CompilerParams", None)
    if cls is None:
        cls = getattr(pltpu, "TPUCompilerParams", None)
    if cls is None:
        return None
    return cls(dimension_semantics=("parallel",))


def _band(w, n_out, n_in):
    T, taps = w.shape[0], w.shape[1]
    rest = w.shape[2:]
    t = jnp.broadcast_to(w[:, None], (T, n_out) + w.shape[1:])
    t = jnp.pad(t, ((0, 0), (0, 0), (0, n_in + 1 - taps))
                + ((0, 0),) * len(rest))
    t = t.reshape((T, n_out * (n_in + 1)) + rest)[:, :n_out * n_in]
    return t.reshape((T, n_out, n_in) + rest)


def _build_w1(w1):
    g = _band(w1, 24, 28)
    g = jnp.transpose(g, (0, 2, 1, 3))
    g = g.reshape(5, 28, 12, 2, 10)
    g = jnp.transpose(g, (3, 4, 2, 0, 1))
    w1a = jnp.pad(g.reshape(240, 140), ((0, 0), (0, 28)))
    w1b = jnp.concatenate([jnp.zeros((240, 28), w1a.dtype),
                           w1a[:, :140]], axis=1)
    return jnp.concatenate([w1a, w1b], axis=0)


def _build_w2(w2):
    w2r = w2.reshape(5, 5, 10, 20)
    g = _band(w2r, 8, 12)
    g = jnp.transpose(g, (0, 2, 1, 3, 4))
    g = g.reshape(5, 12, 4, 2, 10, 20)
    g = jnp.transpose(g, (3, 5, 2, 0, 4, 1))
    w2a = jnp.pad(g.reshape(160, 600), ((0, 0), (0, 120)))
    w2b = jnp.concatenate([jnp.zeros((160, 120), w2a.dtype),
                           w2a[:, :600]], axis=1)
    return jnp.concatenate([w2a, w2b], axis=0)


def _build_wf1(wf1):
    g = wf1.reshape(4, 4, 20, 50)
    g = jnp.transpose(g, (3, 0, 2, 1))
    return g.reshape(50, 320)


def _make_body(BL):
    CB = BL // 128

    def _fused_body(x_ref, w1_ref, w2_ref, wf1_ref, wf2_ref, ball_ref,
                    out_ref, xs_ref, h1_ref, p_ref):
        f32 = jnp.float32
        bf16 = jnp.bfloat16
        X3 = x_ref[...]
        xs_ref[...] = X3.reshape(784, BL).astype(bf16)
        X = xs_ref[...]
        w1ab = w1_ref[...]
        ball = ball_ref[...]
        b1t = ball[0:120, :]

        for h in range(12):
            X6 = X[56 * h:56 * h + 168, :]
            y = jnp.dot(w1ab, X6, preferred_element_type=f32)
            m = jnp.maximum(y[:240, :], y[240:480, :])
            z = jnp.maximum(m[:120, :], m[120:240, :])
            h1_ref[120 * h:120 * h + 120, :] = (
                jnp.maximum(z + b1t, 0.0).astype(bf16))

        w2ab = w2_ref[...]
        b2t = ball[120:200, :]
        for i in range(4):
            H6 = h1_ref[240 * i:240 * i + 720, :]
            y = jnp.dot(w2ab, H6, preferred_element_type=f32)
            m = jnp.maximum(y[:160, :], y[160:320, :])
            z = jnp.maximum(m[:80, :], m[80:160, :])
            p_ref[80 * i:80 * i + 80, :] = (
                jnp.maximum(z + b2t, 0.0).astype(bf16))

        h3 = jnp.dot(wf1_ref[...], p_ref[...], preferred_element_type=f32)
        h3 = jnp.maximum(h3 + ball[200:250, :], 0.0).astype(bf16)
        logits = (jnp.dot(wf2_ref[...], h3, preferred_element_type=f32)
                  + ball[250:260, :])
        mx = jnp.max(logits, axis=0, keepdims=True)
        e = jnp.exp(logits - mx)
        out_ref[...] = logits - mx - jnp.log(jnp.sum(e, axis=0, keepdims=True))

    return _fused_body


def kernel(x, w1, b1, w2, b2, wf1, bf1, wf2, bf2):
    N = x.shape[0]
    Np = (N + 127) // 128 * 128
    xp = jnp.pad(x, ((0, Np - N), (0, 0), (0, 0), (0, 0))) if Np != N else x
    xv = jnp.transpose(xp, (1, 2, 3, 0)).reshape(784, Np // 128, 128)

    BL = 128
    for cand in (1024, 512, 256):
        if Np % cand == 0:
            BL = cand
            break
    CB = BL // 128

    bf16 = jnp.bfloat16
    w1ab = _build_w1(w1).astype(bf16)
    w2ab = _build_w2(w2).astype(bf16)
    wf1b = _build_wf1(wf1).astype(bf16)
    ball = jnp.concatenate([
        jnp.repeat(b1.reshape(10), 12),
        jnp.repeat(b2.reshape(20), 4),
        bf1.reshape(50),
        bf2.reshape(10),
    ]).reshape(260, 1)
    ball = jnp.broadcast_to(ball, (260, BL))
    wf2t = wf2.T.astype(bf16)

    grid_spec = pltpu.PrefetchScalarGridSpec(
        num_scalar_prefetch=0,
        grid=(Np // BL,),
        in_specs=[
            pl.BlockSpec((784, CB, 128), lambda n: (0, n, 0)),
            pl.BlockSpec((480, 168), lambda n: (0, 0)),
            pl.BlockSpec((320, 720), lambda n: (0, 0)),
            pl.BlockSpec((50, 320), lambda n: (0, 0)),
            pl.BlockSpec((10, 50), lambda n: (0, 0)),
            pl.BlockSpec((260, BL), lambda n: (0, 0)),
        ],
        out_specs=pl.BlockSpec((10, BL), lambda n: (0, n)),
        scratch_shapes=[
            pltpu.VMEM((784, BL), jnp.bfloat16),
            pltpu.VMEM((1440, BL), jnp.bfloat16),
            pltpu.VMEM((320, BL), jnp.bfloat16),
        ],
    )
    out = pl.pallas_call(
        _make_body(BL),
        out_shape=jax.ShapeDtypeStruct((10, Np), jnp.float32),
        grid_spec=grid_spec,
        compiler_params=_params_for_tc(),
    )(xv, w1ab, w2ab, wf1b, wf2t, ball)
    return out[:, :N].T if Np != N else out.T
